# baseline (speedup 1.0000x reference)
import jax
import jax.numpy as jnp
from jax import lax
from jax.experimental import layout
from jax.experimental import pallas as pl
from jax.experimental.pallas import tpu as pltpu
from jax.experimental.pallas import tpu_sc as plsc

NC, NS, L = 2, 16, 16
NW = NC * NS
B, S, M, D = 4096, 50, 5, 64
BATCHES_W = B // NW
CB = 2
CHUNK = CB * S * M
NCHUNK = BATCHES_W // CB
PAIR = 2 * CHUNK
NBUF = 2
PIECE = 128
EVEN_SPLITS = ((0, 0), (128, 128), (256, 256), (376, 376))
ODD_SPLITS = ((496, 0), (624, 128), (744, 248), (872, 376))
NROWS = 512


def _body(idx_hbm, table_hbm, out3_hbm, idx_v, rows_v, out_v, sem0, sem1):
  sems = (sem0, sem1)
  wid = lax.axis_index("s") * NC + lax.axis_index("c")
  wb0 = wid * BATCHES_W
  wi0 = wid * BATCHES_W * S * M

  def fire(b, c, ps):
    if b == 0:
      pltpu.sync_copy(
          idx_hbm.at[pl.ds(wi0 + c * CHUNK, PAIR)],
          idx_v.at[ps],
      )
    splits = EVEN_SPLITS if b == 0 else ODD_SPLITS
    for soff, doff in splits:
      pltpu.async_copy(
          table_hbm.at[idx_v.at[ps, pl.ds(soff, PIECE)]],
          rows_v.at[b, pl.ds(doff, PIECE)],
          sems[b],
      )

  def drain(b):
    pltpu.make_async_copy(
        table_hbm.at[pl.ds(0, NROWS)], rows_v.at[b], sems[b]
    ).wait()

  def reduce_store(b, c):
    ro = 0 if b == 0 else 4

    for gb in range(CB):
      def grp(ss, carry, gb=gb):
        r = ro + (gb * S + ss) * M
        for d in range(D // L):
          sl = pl.ds(d * L, L)
          acc = rows_v[b, r, sl]
          for m in range(1, M):
            acc = acc + rows_v[b, r + m, sl]
          out_v[gb, ss, sl] = acc * (1.0 / M)
        return carry

      lax.fori_loop(0, S, grp, 0, unroll=2)
    pltpu.sync_copy(out_v, out3_hbm.at[pl.ds(wb0 + c * CB, CB)])

  fire(0, 0, 0)
  fire(1, 1, 0)

  def step(s, carry):
    pn = (s + 1) & 1
    for b in range(NBUF):
      c = s * NBUF + b
      drain(b)
      reduce_store(b, c)
      cn = c + NBUF

      @pl.when(cn < NCHUNK)
      def _():
        fire(b, cn, pn)

    return carry

  lax.fori_loop(0, NCHUNK // NBUF, step, 0)


_sc_call = pl.kernel(
    _body,
    out_type=jax.ShapeDtypeStruct((B, S, D), jnp.float32),
    mesh=plsc.VectorSubcoreMesh(
        core_axis_name="c", subcore_axis_name="s", num_cores=NC,
        num_subcores=NS),
    scratch_types=[
        pltpu.VMEM((2, PAIR), jnp.int32),
        pltpu.VMEM((NBUF, NROWS, D), jnp.float32),
        pltpu.VMEM((CB, S, D), jnp.float32),
        pltpu.SemaphoreType.DMA,
        pltpu.SemaphoreType.DMA,
    ],
    compiler_params=pltpu.CompilerParams(use_tc_tiling_on_sc=False),
)


_state = {}


def _get_run(sharding):
  run = _state.get("run")
  if run is None:
    out_fmt = layout.Format(
        layout.Layout(major_to_minor=(0, 1, 2)), sharding)
    run = jax.jit(
        lambda meta, tp: _sc_call(
            meta.astype(jnp.int32).reshape(B * S * M), tp),
        out_shardings=out_fmt)
    _state["run"] = run
  return run


def _prep_table(table):
  cached = _state.get("table")
  if cached is not None and cached[0] is table:
    return cached[1]
  fmt = layout.Format(layout.Layout(major_to_minor=(0, 1)), table.sharding)
  tp = jax.device_put(table, fmt)
  _state["table"] = (table, tp)
  return tp


def kernel(meta_indices, table):
  if isinstance(table, jax.core.Tracer):
    return _sc_call(meta_indices.astype(jnp.int32).reshape(B * S * M), table)
  run = _get_run(table.sharding)
  return run(meta_indices, _prep_table(table))


# speedup vs baseline: 1.0585x; 1.0585x over previous
import jax
import jax.numpy as jnp
from jax import lax
from jax.experimental import pallas as pl
from jax.experimental.pallas import tpu as pltpu
from jax.experimental.pallas import tpu_sc as plsc

NC, NS, L = 2, 16, 16
NW = NC * NS
B, S, M, D = 4096, 50, 5, 64
DP = 128
BATCHES_W = B // NW
CHUNK = S * M
NCHUNK = BATCHES_W
QUAD = 4 * CHUNK
NBUF = 2
PIECE = 128
NROWS = 256
N = B * S * M


def _body(idx_hbm, table_hbm, out3_hbm, idx_v, rows_v, out_v, sem0, sem1):
  sems = (sem0, sem1)
  wid = lax.axis_index("s") * NC + lax.axis_index("c")
  wb0 = wid * BATCHES_W
  wi0 = wid * BATCHES_W * CHUNK

  def load_quad(q, qs):
    pltpu.sync_copy(idx_hbm.at[pl.ds(wi0 + q * QUAD, QUAD)], idx_v.at[qs])

  def fire(b, k, qs):
    for j in range(2):
      soff = 248 * k + j * PIECE
      pltpu.async_copy(
          table_hbm.at[idx_v.at[qs, pl.ds(soff, PIECE)]],
          rows_v.at[b, pl.ds(j * PIECE, PIECE)],
          sems[b],
      )

  def drain(b):
    pltpu.make_async_copy(
        table_hbm.at[pl.ds(0, NROWS)], rows_v.at[b], sems[b]
    ).wait()

  def reduce_store(b, c, k):
    ro = 2 * k

    def grp(ss, carry):
      r = ro + ss * M
      for d in range(D // L):
        sl = pl.ds(d * L, L)
        acc = rows_v[b, r, sl]
        for m in range(1, M):
          acc = acc + rows_v[b, r + m, sl]
        out_v[ss, sl] = acc * (1.0 / M)
      return carry

    lax.fori_loop(0, S, grp, 0, unroll=2)
    pltpu.sync_copy(out_v, out3_hbm.at[wb0 + c])

  load_quad(0, 0)
  fire(0, 0, 0)
  fire(1, 1, 0)

  def step(s, carry):
    qs_cur = s & 1
    qs_next = (s + 1) & 1
    for k in range(4):
      b = k & 1
      c = 4 * s + k
      drain(b)
      reduce_store(b, c, k)
      cn = c + NBUF
      kn = (k + NBUF) & 3
      qn = qs_next if k >= 2 else qs_cur

      @pl.when(cn < NCHUNK)
      def _():
        if k == 2:
          load_quad(s + 1, qs_next)
        fire(b, kn, qn)

    return carry

  lax.fori_loop(0, NCHUNK // 4, step, 0)


_sc_call = pl.kernel(
    _body,
    out_type=jax.ShapeDtypeStruct((B, S, D), jnp.float32),
    mesh=plsc.VectorSubcoreMesh(
        core_axis_name="c", subcore_axis_name="s", num_cores=NC,
        num_subcores=NS),
    scratch_types=[
        pltpu.VMEM((2, QUAD), jnp.int32),
        pltpu.VMEM((NBUF, NROWS, DP), jnp.float32),
        pltpu.VMEM((S, D), jnp.float32),
        pltpu.SemaphoreType.DMA,
        pltpu.SemaphoreType.DMA,
    ],
    compiler_params=pltpu.CompilerParams(use_tc_tiling_on_sc=False),
)


def kernel(meta_indices, table):
  table_p = jnp.pad(table, ((0, 0), (0, DP - D)))
  return _sc_call(meta_indices.astype(jnp.int32).reshape(N), table_p)
